# baseline (speedup 1.0000x reference)
import functools

import jax
import jax.numpy as jnp
from jax import lax
from jax.experimental import pallas as pl
from jax.experimental.pallas import tpu as pltpu
from jax.experimental.pallas import tpu_sc as plsc

B = 4096
L = 50
D = 16
LP = 64
NC_PAD = 1024
NR = 5
NR_PAD = 16
NW = 32
ROWS = B // NW
GRP = 4
NG = ROWS // GRP


def _tc_precompute(c2e_p, r2e_p, Wk, Wv, Wo, bv2, bo2):
    def body(c2e_ref, r2e_ref, wk_ref, wv_ref, wo_ref, bv_ref, bo_ref,
             skt_ref, rkt_ref, svr_ref):
        c2e = c2e_ref[...]
        r2e = r2e_ref[...]
        wk0 = wk_ref[0:D, :]
        wk1 = wk_ref[D:2 * D, :]
        wv0 = wv_ref[0:D, :]
        wv1 = wv_ref[D:2 * D, :]
        wo = wo_ref[...]
        scale = 0.25
        sk = jnp.dot(c2e, wk0, preferred_element_type=jnp.float32) * scale
        skt_ref[...] = sk.T
        rk = jnp.dot(r2e, wk1, preferred_element_type=jnp.float32) * scale
        rkt_ref[...] = rk.T
        wvo0 = jnp.dot(wv0, wo, preferred_element_type=jnp.float32)
        wvo1 = jnp.dot(wv1, wo, preferred_element_type=jnp.float32)
        cb = jnp.dot(bv_ref[...], wo, preferred_element_type=jnp.float32) + bo_ref[...]
        sv = jnp.dot(c2e, wvo0, preferred_element_type=jnp.float32)
        rvp = jnp.dot(r2e, wvo1, preferred_element_type=jnp.float32) + cb
        for r in range(NR):
            svr_ref[:, r * D:(r + 1) * D] = sv + rvp[r, :]

    return pl.pallas_call(
        body,
        out_shape=(
            jax.ShapeDtypeStruct((D, NC_PAD), jnp.float32),
            jax.ShapeDtypeStruct((D, NR_PAD), jnp.float32),
            jax.ShapeDtypeStruct((NC_PAD, NR * D), jnp.float32),
        ),
    )(c2e_p, r2e_p, Wk, Wv, Wo, bv2, bo2)


def _sc_fused(v2eT, hcomb2, skt, rkt_p, svr1d, wq_p, bq_p):
    mesh = plsc.VectorSubcoreMesh(core_axis_name="c", subcore_axis_name="s")

    @functools.partial(
        pl.kernel,
        mesh=mesh,
        compiler_params=pltpu.CompilerParams(
            needs_layout_passes=False, use_tc_tiling_on_sc=True),
        out_type=jax.ShapeDtypeStruct((B // 8, 128), jnp.float32),
        scratch_types=[
            pltpu.VMEM((D, NC_PAD), jnp.float32),
            pltpu.VMEM((D, 128), jnp.float32),
            pltpu.VMEM((NC_PAD * NR * D,), jnp.float32),
            pltpu.VMEM((D, 128), jnp.float32),
            pltpu.VMEM((8, 128), jnp.float32),
            pltpu.VMEM((ROWS // 2, 128), jnp.int32),
            pltpu.VMEM((GRP, D, 128), jnp.float32),
            pltpu.VMEM((GRP, D, 128), jnp.float32),
            pltpu.VMEM((GRP, D), jnp.float32),
            pltpu.VMEM((ROWS * D // 128, 128), jnp.float32),
            pltpu.SemaphoreType.DMA,
            pltpu.SemaphoreType.DMA,
            pltpu.SemaphoreType.DMA,
        ],
    )
    def k(v2eT_hbm, hcomb_hbm, skt_hbm, rkt_hbm, svr_hbm, wq_hbm, bq_hbm,
          out_hbm, skt_v, rkt_v, svr_v, wq_v, bq_v, hcomb_v,
          waveA, waveB, rkq_buf, outbuf, semA, semB, semS):
        wid = lax.axis_index("c") * 16 + lax.axis_index("s")

        pltpu.sync_copy(hcomb_hbm.at[pl.ds(wid * (ROWS // 2), ROWS // 2)],
                        hcomb_v)
        staging = [
            pltpu.async_copy(skt_hbm, skt_v, semS),
            pltpu.async_copy(rkt_hbm, rkt_v, semS),
            pltpu.async_copy(svr_hbm, svr_v, semS),
            pltpu.async_copy(wq_hbm, wq_v, semS),
            pltpu.async_copy(bq_hbm, bq_v, semS),
        ]

        iota = lax.iota(jnp.int32, 16)
        lanemask_last = iota < (L - 3 * 16)
        neg = jnp.full((16,), -1e30, jnp.float32)
        nchunk = LP // 16
        m127 = jnp.full((16,), 127, jnp.int32)
        c7 = jnp.full((16,), 7, jnp.int32)

        def nodevec_of(g, j):
            i = g * GRP + j
            return hcomb_v[lax.div(i, 2), pl.ds((j % 2) * 64 + 48, 16)]

        def issue(g, wave, sem):
            for j in range(GRP):
                nv = nodevec_of(g, j)
                hiv = jnp.right_shift(nv, c7)
                pltpu.async_copy(
                    v2eT_hbm.at[:, pl.ds(hiv[15] * 128, 128)], wave.at[j], sem)

        def drain(wave, sem):
            for j in range(GRP):
                pltpu.make_async_copy(
                    v2eT_hbm.at[:, pl.ds(0, 128)], wave.at[j], sem).wait()

        def one_row(g, j, wave):
            i = g * GRP + j
            nv = nodevec_of(g, j)
            modv = jnp.bitwise_and(nv, m127)
            vcvec = plsc.load_gather(
                wave, [jnp.full((16,), j, jnp.int32), iota,
                       jnp.broadcast_to(modv[15], (16,))])
            qa = [bq_v[0, 0:16], jnp.zeros((16,), jnp.float32),
                  jnp.zeros((16,), jnp.float32), jnp.zeros((16,), jnp.float32)]
            for d in range(D):
                qa[d % 4] = qa[d % 4] + vcvec[d] * wq_v[d, 0:16]
            q = (qa[0] + qa[1]) + (qa[2] + qa[3])
            qs = [q[d] for d in range(D)]
            ra = [jnp.zeros((16,), jnp.float32) for _ in range(4)]
            for d in range(D):
                ra[d % 4] = ra[d % 4] + qs[d] * rkt_v[d, 0:16]
            rkq_buf[j, :] = (ra[0] + ra[1]) + (ra[2] + ra[3])
            slotv = jnp.full((16,), j, jnp.int32)
            row2 = lax.div(i, 2)
            lane0 = (j % 2) * 64
            chunks, flats = [], []
            for t in range(nchunk):
                packed = hcomb_v[row2, pl.ds(lane0 + 16 * t, 16)]
                cv = jnp.bitwise_and(packed, jnp.full((16,), NC_PAD - 1, jnp.int32))
                rr = jnp.bitwise_and(
                    jnp.right_shift(packed, jnp.full((16,), 10, jnp.int32)),
                    jnp.full((16,), 15, jnp.int32))
                flats.append((cv * NR + rr) * D)
                sa = [plsc.load_gather(rkq_buf, [slotv, rr]),
                      jnp.zeros((16,), jnp.float32),
                      jnp.zeros((16,), jnp.float32),
                      jnp.zeros((16,), jnp.float32)]
                for d in range(D):
                    dvec = jnp.full((16,), d, jnp.int32)
                    sa[d % 4] = sa[d % 4] + qs[d] * plsc.load_gather(
                        skt_v, [dvec, cv])
                chunks.append((sa[0] + sa[1]) + (sa[2] + sa[3]))
            chunks[3] = jnp.where(lanemask_last, chunks[3], neg)
            m = jnp.max(jnp.maximum(jnp.maximum(chunks[0], chunks[1]),
                                    jnp.maximum(chunks[2], chunks[3])))
            es = [jnp.exp(c - m) for c in chunks]
            total = jnp.sum((es[0] + es[1]) + (es[2] + es[3]))
            inv = jnp.full((16,), 1.0, jnp.float32) / jnp.broadcast_to(
                total, (16,))
            oa = [jnp.zeros((16,), jnp.float32) for _ in range(4)]
            for t in range(nchunk):
                at = es[t] * inv
                for jj in range(16):
                    l = 16 * t + jj
                    if l >= L:
                        break
                    fidx = jnp.broadcast_to(flats[t][jj], (16,)) + iota
                    row = plsc.load_gather(svr_v, [fidx])
                    oa[l % 4] = oa[l % 4] + at[jj] * row
            out = (oa[0] + oa[1]) + (oa[2] + oa[3])
            iv = jnp.full((16,), i, jnp.int32)
            orow = jnp.right_shift(iv, jnp.full((16,), 3, jnp.int32))
            ocol = jnp.bitwise_and(iv, jnp.full((16,), 7, jnp.int32)) * D + iota
            plsc.store_scatter(outbuf, [orow, ocol], out)

        def process(g, wave):
            for j in range(GRP):
                one_row(g, j, wave)

        issue(0, waveA, semA)
        for cp in staging:
            cp.wait()

        def body(h, carry):
            g0 = 2 * h
            g1 = g0 + 1
            issue(g1, waveB, semB)
            drain(waveA, semA)
            process(g0, waveA)
            issue(jnp.minimum(g0 + 2, NG - 1), waveA, semA)
            drain(waveB, semB)
            process(g1, waveB)
            return carry

        lax.fori_loop(0, NG // 2, body, 0)
        drain(waveA, semA)
        pltpu.sync_copy(outbuf, out_hbm.at[pl.ds(wid * (ROWS * D // 128),
                                                 ROWS * D // 128)])

    return k(v2eT, hcomb2, skt, rkt_p, svr1d, wq_p, bq_p)


def kernel(nodes, history_vc, history_r, c2e_weight, r2e_weight, v2e_weight,
           Wq, bq, Wk, bk, Wv, bv, Wo, bo):
    nodes = nodes.astype(jnp.int32)
    v2eT = v2e_weight.T
    packed = (history_r.astype(jnp.int32) << 10) | history_vc.astype(jnp.int32)
    hcomb = jnp.concatenate(
        [packed, jnp.zeros((B, LP - L - 1), jnp.int32), nodes.reshape(B, 1)],
        axis=1)
    hcomb2 = hcomb.reshape(B * LP // 128, 128)
    c2e_p = jnp.pad(c2e_weight, ((0, NC_PAD - c2e_weight.shape[0]), (0, 0)))
    r2e_p = jnp.pad(r2e_weight, ((0, NR_PAD - r2e_weight.shape[0]), (0, 0)))
    bv2 = bv.reshape(1, D)
    bo2 = bo.reshape(1, D)
    skt, rkt, svr80 = _tc_precompute(c2e_p, r2e_p, Wk, Wv, Wo, bv2, bo2)
    rkt_p = jnp.pad(rkt, ((0, 0), (0, 128 - NR_PAD)))
    svr1d = svr80.reshape(NC_PAD * NR * D)
    wq_p = jnp.pad(Wq, ((0, 0), (0, 128 - D)))
    bq_p = jnp.pad(bq.reshape(1, D), ((0, 7), (0, 128 - D)))
    out = _sc_fused(v2eT, hcomb2, skt, rkt_p, svr1d, wq_p, bq_p)
    return out.reshape(B, D)


# speedup vs baseline: 1.2294x; 1.2294x over previous
import functools

import jax
import jax.numpy as jnp
from jax import lax
from jax.experimental import pallas as pl
from jax.experimental.pallas import tpu as pltpu
from jax.experimental.pallas import tpu_sc as plsc

B = 4096
L = 50
D = 16
LP = 64
NC_PAD = 1024
NR = 5
NR_PAD = 16
NW = 32
ROWS = B // NW
GRP = 2
NG = ROWS // GRP


def _tc_precompute(c2e_p, r2e_p, Wk, Wv, Wo, bv2, bo2):
    def body(c2e_ref, r2e_ref, wk_ref, wv_ref, wo_ref, bv_ref, bo_ref,
             skt_ref, rkt_ref, svr_ref):
        c2e = c2e_ref[...]
        r2e = r2e_ref[...]
        wk0 = wk_ref[0:D, :]
        wk1 = wk_ref[D:2 * D, :]
        wv0 = wv_ref[0:D, :]
        wv1 = wv_ref[D:2 * D, :]
        wo = wo_ref[...]
        scale = 0.25
        sk = jnp.dot(c2e, wk0, preferred_element_type=jnp.float32) * scale
        skt_ref[...] = sk.T
        rk = jnp.dot(r2e, wk1, preferred_element_type=jnp.float32) * scale
        rkt_ref[...] = rk.T
        wvo0 = jnp.dot(wv0, wo, preferred_element_type=jnp.float32)
        wvo1 = jnp.dot(wv1, wo, preferred_element_type=jnp.float32)
        cb = jnp.dot(bv_ref[...], wo, preferred_element_type=jnp.float32) + bo_ref[...]
        sv = jnp.dot(c2e, wvo0, preferred_element_type=jnp.float32)
        rvp = jnp.dot(r2e, wvo1, preferred_element_type=jnp.float32) + cb
        for r in range(NR):
            svr_ref[:, r * D:(r + 1) * D] = sv + rvp[r, :]

    return pl.pallas_call(
        body,
        out_shape=(
            jax.ShapeDtypeStruct((D, NC_PAD), jnp.float32),
            jax.ShapeDtypeStruct((D, NR_PAD), jnp.float32),
            jax.ShapeDtypeStruct((NC_PAD, NR * D), jnp.float32),
        ),
    )(c2e_p, r2e_p, Wk, Wv, Wo, bv2, bo2)


def _sc_fused(v2eT, hcomb2, skt, rkt_p, svr1d, wq_p, bq_p):
    mesh = plsc.VectorSubcoreMesh(core_axis_name="c", subcore_axis_name="s")

    @functools.partial(
        pl.kernel,
        mesh=mesh,
        compiler_params=pltpu.CompilerParams(
            needs_layout_passes=False, use_tc_tiling_on_sc=True),
        out_type=jax.ShapeDtypeStruct((B // 8, 128), jnp.float32),
        scratch_types=[
            pltpu.VMEM((D, NC_PAD), jnp.float32),
            pltpu.VMEM((D, 128), jnp.float32),
            pltpu.VMEM((NC_PAD * NR * D,), jnp.float32),
            pltpu.VMEM((D, 128), jnp.float32),
            pltpu.VMEM((8, 128), jnp.float32),
            pltpu.VMEM((ROWS // 2, 128), jnp.int32),
            pltpu.VMEM((GRP, D, 128), jnp.float32),
            pltpu.VMEM((GRP, D, 128), jnp.float32),
            pltpu.VMEM((GRP, D), jnp.float32),
            pltpu.VMEM((ROWS * D // 128, 128), jnp.float32),
            pltpu.SemaphoreType.DMA,
            pltpu.SemaphoreType.DMA,
            pltpu.SemaphoreType.DMA,
        ],
    )
    def k(v2eT_hbm, hcomb_hbm, skt_hbm, rkt_hbm, svr_hbm, wq_hbm, bq_hbm,
          out_hbm, skt_v, rkt_v, svr_v, wq_v, bq_v, hcomb_v,
          waveA, waveB, rkq_buf, outbuf, semA, semB, semS):
        wid = lax.axis_index("c") * 16 + lax.axis_index("s")

        pltpu.sync_copy(hcomb_hbm.at[pl.ds(wid * (ROWS // 2), ROWS // 2)],
                        hcomb_v)
        staging = [
            pltpu.async_copy(skt_hbm, skt_v, semS),
            pltpu.async_copy(rkt_hbm, rkt_v, semS),
            pltpu.async_copy(svr_hbm, svr_v, semS),
            pltpu.async_copy(wq_hbm, wq_v, semS),
            pltpu.async_copy(bq_hbm, bq_v, semS),
        ]

        iota = lax.iota(jnp.int32, 16)
        lanemask_last = iota < (L - 3 * 16)
        neg = jnp.full((16,), -1e30, jnp.float32)
        nchunk = LP // 16
        m127 = jnp.full((16,), 127, jnp.int32)
        c7 = jnp.full((16,), 7, jnp.int32)

        def nodevec_of(g, j):
            i = g * GRP + j
            return hcomb_v[lax.div(i, 2), pl.ds((j % 2) * 64 + 48, 16)]

        def issue(g, wave, sem):
            for j in range(GRP):
                nv = nodevec_of(g, j)
                hiv = jnp.right_shift(nv, c7)
                pltpu.async_copy(
                    v2eT_hbm.at[:, pl.ds(hiv[15] * 128, 128)], wave.at[j], sem)

        def drain(wave, sem):
            for j in range(GRP):
                pltpu.make_async_copy(
                    v2eT_hbm.at[:, pl.ds(0, 128)], wave.at[j], sem).wait()

        def one_row(g, j, wave):
            i = g * GRP + j
            nv = nodevec_of(g, j)
            modv = jnp.bitwise_and(nv, m127)
            vcvec = plsc.load_gather(
                wave, [jnp.full((16,), j, jnp.int32), iota,
                       jnp.broadcast_to(modv[15], (16,))])
            qa = [bq_v[0, 0:16], jnp.zeros((16,), jnp.float32),
                  jnp.zeros((16,), jnp.float32), jnp.zeros((16,), jnp.float32)]
            for d in range(D):
                qa[d % 4] = qa[d % 4] + vcvec[d] * wq_v[d, 0:16]
            q = (qa[0] + qa[1]) + (qa[2] + qa[3])
            qs = [q[d] for d in range(D)]
            ra = [jnp.zeros((16,), jnp.float32) for _ in range(4)]
            for d in range(D):
                ra[d % 4] = ra[d % 4] + qs[d] * rkt_v[d, 0:16]
            rkq_buf[j, :] = (ra[0] + ra[1]) + (ra[2] + ra[3])
            slotv = jnp.full((16,), j, jnp.int32)
            row2 = lax.div(i, 2)
            lane0 = (j % 2) * 64
            chunks, flats = [], []
            for t in range(nchunk):
                packed = hcomb_v[row2, pl.ds(lane0 + 16 * t, 16)]
                cv = jnp.bitwise_and(packed, jnp.full((16,), NC_PAD - 1, jnp.int32))
                rr = jnp.bitwise_and(
                    jnp.right_shift(packed, jnp.full((16,), 10, jnp.int32)),
                    jnp.full((16,), 15, jnp.int32))
                flats.append((cv * NR + rr) * D)
                sa = [plsc.load_gather(rkq_buf, [slotv, rr]),
                      jnp.zeros((16,), jnp.float32),
                      jnp.zeros((16,), jnp.float32),
                      jnp.zeros((16,), jnp.float32)]
                for d in range(D):
                    dvec = jnp.full((16,), d, jnp.int32)
                    sa[d % 4] = sa[d % 4] + qs[d] * plsc.load_gather(
                        skt_v, [dvec, cv])
                chunks.append((sa[0] + sa[1]) + (sa[2] + sa[3]))
            chunks[3] = jnp.where(lanemask_last, chunks[3], neg)
            m = jnp.max(jnp.maximum(jnp.maximum(chunks[0], chunks[1]),
                                    jnp.maximum(chunks[2], chunks[3])))
            es = [jnp.exp(c - m) for c in chunks]
            total = jnp.sum((es[0] + es[1]) + (es[2] + es[3]))
            inv = jnp.full((16,), 1.0, jnp.float32) / jnp.broadcast_to(
                total, (16,))
            oa = [jnp.zeros((16,), jnp.float32) for _ in range(4)]
            for t in range(nchunk):
                at = es[t] * inv
                for jj in range(16):
                    l = 16 * t + jj
                    if l >= L:
                        break
                    fidx = jnp.broadcast_to(flats[t][jj], (16,)) + iota
                    row = plsc.load_gather(svr_v, [fidx])
                    oa[l % 4] = oa[l % 4] + at[jj] * row
            out = (oa[0] + oa[1]) + (oa[2] + oa[3])
            iv = jnp.full((16,), i, jnp.int32)
            orow = jnp.right_shift(iv, jnp.full((16,), 3, jnp.int32))
            ocol = jnp.bitwise_and(iv, jnp.full((16,), 7, jnp.int32)) * D + iota
            plsc.store_scatter(outbuf, [orow, ocol], out)

        def process(g, wave):
            for j in range(GRP):
                one_row(g, j, wave)

        issue(0, waveA, semA)
        for cp in staging:
            cp.wait()

        def body(h, carry):
            g0 = 2 * h
            g1 = g0 + 1
            issue(g1, waveB, semB)
            drain(waveA, semA)
            process(g0, waveA)
            issue(jnp.minimum(g0 + 2, NG - 1), waveA, semA)
            drain(waveB, semB)
            process(g1, waveB)
            return carry

        lax.fori_loop(0, NG // 2, body, 0)
        drain(waveA, semA)
        pltpu.sync_copy(outbuf, out_hbm.at[pl.ds(wid * (ROWS * D // 128),
                                                 ROWS * D // 128)])

    return k(v2eT, hcomb2, skt, rkt_p, svr1d, wq_p, bq_p)


def kernel(nodes, history_vc, history_r, c2e_weight, r2e_weight, v2e_weight,
           Wq, bq, Wk, bk, Wv, bv, Wo, bo):
    nodes = nodes.astype(jnp.int32)
    v2eT = v2e_weight.T
    packed = (history_r.astype(jnp.int32) << 10) | history_vc.astype(jnp.int32)
    hcomb = jnp.concatenate(
        [packed, jnp.zeros((B, LP - L - 1), jnp.int32), nodes.reshape(B, 1)],
        axis=1)
    hcomb2 = hcomb.reshape(B * LP // 128, 128)
    c2e_p = jnp.pad(c2e_weight, ((0, NC_PAD - c2e_weight.shape[0]), (0, 0)))
    r2e_p = jnp.pad(r2e_weight, ((0, NR_PAD - r2e_weight.shape[0]), (0, 0)))
    bv2 = bv.reshape(1, D)
    bo2 = bo.reshape(1, D)
    skt, rkt, svr80 = _tc_precompute(c2e_p, r2e_p, Wk, Wv, Wo, bv2, bo2)
    rkt_p = jnp.pad(rkt, ((0, 0), (0, 128 - NR_PAD)))
    svr1d = svr80.reshape(NC_PAD * NR * D)
    wq_p = jnp.pad(Wq, ((0, 0), (0, 128 - D)))
    bq_p = jnp.pad(bq.reshape(1, D), ((0, 7), (0, 128 - D)))
    out = _sc_fused(v2eT, hcomb2, skt, rkt_p, svr1d, wq_p, bq_p)
    return out.reshape(B, D)
